# baseline (speedup 1.0000x reference)
import jax
import jax.numpy as jnp
from jax import lax
from jax.experimental import pallas as pl
from jax.experimental.pallas import tpu as pltpu
from jax.experimental.pallas import tpu_sc as plsc

_NC, _NS = 2, 16
_NW = _NC * _NS


def _col_fix_body(x_hbm, cg_hbm, out_hbm, x_v, cg_v, o_v):
    nch = out_hbm.shape[1]
    wid = lax.axis_index("s") * _NC + lax.axis_index("c")
    pltpu.sync_copy(x_hbm.at[wid], x_v)
    pltpu.sync_copy(cg_hbm.at[wid], cg_v)
    pred = x_v[...] > 0.5
    for i in range(nch):
        g0 = cg_v[2 * i, :]
        if i < nch - 1:
            g = jnp.where(pred, cg_v[2 * i + 1, :], g0)
        else:
            g = g0
        o_v[i, :] = 1.0 / (1.0 + jnp.exp(-g))
    pltpu.sync_copy(o_v, out_hbm.at[wid])


def _col_fix(xp, cgp, nch, rows):
    return pl.kernel(
        _col_fix_body,
        out_type=jax.ShapeDtypeStruct((_NW, nch, rows), jnp.float32),
        mesh=plsc.VectorSubcoreMesh(core_axis_name="c", subcore_axis_name="s"),
        scratch_types=[
            pltpu.VMEM((rows,), jnp.float32),
            pltpu.VMEM((2 * nch, rows), jnp.float32),
            pltpu.VMEM((nch, rows), jnp.float32),
        ],
    )(xp, cgp)


def _gate_kernel(tg0_ref, corr_ref, mask_ref, out_ref):
    n = out_ref.shape[1]
    dense = tg0_ref[0, 0]
    corr = corr_ref[0]
    is_col0 = jax.lax.broadcasted_iota(jnp.int32, (n, n), 1) == 0
    val = jnp.where(is_col0, corr, jax.nn.sigmoid(dense))
    out_ref[0] = jnp.where(mask_ref[0], val, 0.0)


def kernel(x, mask, toggle_gates):
    c, _, n, _ = toggle_gates.shape
    rows = n // _NW
    cgp = toggle_gates[:, 0:2, :, 0].reshape(2 * c, _NW, rows).transpose(1, 0, 2)
    xp = x.reshape(_NW, rows)
    corr_w = _col_fix(xp, cgp, c, rows)
    corr = corr_w.transpose(1, 0, 2).reshape(c, n)
    mask3 = mask.reshape(c, n, n)
    out = pl.pallas_call(
        _gate_kernel,
        grid=(c,),
        in_specs=[
            pl.BlockSpec((1, 1, n, n), lambda i: (i, 0, 0, 0)),
            pl.BlockSpec((1, n, 1), lambda i: (i, 0, 0)),
            pl.BlockSpec((1, n, n), lambda i: (i, 0, 0)),
        ],
        out_shape=jax.ShapeDtypeStruct((c, n, n), jnp.float32),
        out_specs=pl.BlockSpec((1, n, n), lambda i: (i, 0, 0)),
    )(toggle_gates, corr.reshape(c, n, 1), mask3)
    return out.reshape(-1)


# speedup vs baseline: 1.0083x; 1.0083x over previous
import jax
import jax.numpy as jnp
from jax import lax
from jax.experimental import pallas as pl
from jax.experimental.pallas import tpu as pltpu
from jax.experimental.pallas import tpu_sc as plsc

_NC, _NS = 2, 16
_NW = _NC * _NS


def _col_fix(xp, cgp, nch, n, rows):
    def body(x_hbm, cg_hbm, out_hbm, x_v, cg_v, o_v):
        wid = lax.axis_index("s") * _NC + lax.axis_index("c")
        base = wid * rows
        pltpu.sync_copy(x_hbm.at[wid], x_v)
        pltpu.sync_copy(cg_hbm.at[wid], cg_v)
        pred = x_v[...] > 0.5
        for i in range(nch):
            g0 = cg_v[2 * i, :]
            if i < nch - 1:
                g = jnp.where(pred, cg_v[2 * i + 1, :], g0)
            else:
                g = g0
            o_v[i, :] = 1.0 / (1.0 + jnp.exp(-g))
        for i in range(nch):
            pltpu.sync_copy(o_v.at[i], out_hbm.at[i, pl.ds(base, rows)])

    return pl.kernel(
        body,
        out_type=jax.ShapeDtypeStruct((nch, n), jnp.float32),
        mesh=plsc.VectorSubcoreMesh(core_axis_name="c", subcore_axis_name="s"),
        scratch_types=[
            pltpu.VMEM((rows,), jnp.float32),
            pltpu.VMEM((2 * nch, rows), jnp.float32),
            pltpu.VMEM((nch, rows), jnp.float32),
        ],
    )(xp, cgp)


def _gate_kernel(tg0_ref, corr_ref, mask_ref, out_ref):
    n = out_ref.shape[1]
    dense = tg0_ref[0, 0]
    corr = corr_ref[0]
    is_col0 = jax.lax.broadcasted_iota(jnp.int32, (n, n), 1) == 0
    val = jnp.where(is_col0, corr, jax.nn.sigmoid(dense))
    out_ref[0] = jnp.where(mask_ref[0], val, 0.0)


def kernel(x, mask, toggle_gates):
    c, _, n, _ = toggle_gates.shape
    rows = n // _NW
    xp = x.reshape(_NW, rows)
    cgp = toggle_gates[:, 0:2, :, 0].reshape(2 * c, _NW, rows).transpose(1, 0, 2)
    corr = _col_fix(xp, cgp, c, n, rows)
    mask3 = mask.reshape(c, n, n)
    out = pl.pallas_call(
        _gate_kernel,
        grid=(c,),
        in_specs=[
            pl.BlockSpec((1, 1, n, n), lambda i: (i, 0, 0, 0)),
            pl.BlockSpec((1, n, 1), lambda i: (i, 0, 0)),
            pl.BlockSpec((1, n, n), lambda i: (i, 0, 0)),
        ],
        out_shape=jax.ShapeDtypeStruct((c, n, n), jnp.float32),
        out_specs=pl.BlockSpec((1, n, n), lambda i: (i, 0, 0)),
    )(toggle_gates, corr.reshape(c, n, 1), mask3)
    return out.reshape(-1)
